# baseline (speedup 1.0000x reference)
import functools

import jax
import jax.numpy as jnp
from jax import lax
from jax.experimental import pallas as pl
from jax.experimental.pallas import tpu as pltpu
from jax.experimental.pallas import tpu_sc as plsc

F32 = jnp.float32
BF16 = jnp.bfloat16
I32 = jnp.int32

N_TOK = 2048
D_MODEL = 1024
DH = D_MODEL // 2
N_EXP = 8
D_EXP = 256
D_SHARED = 512
EP = 128
M_TILE = 256
N_TILES = 24
N_SLOT = N_TILES * M_TILE
RB = 1024
N_RB = N_TOK // RB
CB = 1024
N_CB = N_TOK // CB
NUM_SC_CORES = 2
NUM_SC_SUBCORES = 16
N_WORKERS = NUM_SC_CORES * NUM_SC_SUBCORES
TPW = N_TOK // N_WORKERS


def _pack_bf16(v):
    lo = lax.bitcast_convert_type(v[:, :DH], jnp.int16).astype(I32) & 0xFFFF
    hi = lax.bitcast_convert_type(v[:, DH:], jnp.int16).astype(I32)
    return lax.shift_left(hi, 16) | lo


def _unpack_bf16(p):
    lo = lax.bitcast_convert_type(p.astype(jnp.int16), BF16)
    hi = lax.bitcast_convert_type(
        lax.shift_right_logical(p, 16).astype(jnp.int16), BF16)
    return jnp.concatenate([lo, hi], axis=1)


def _router_body(x_ref, wg_ref, meta_ref, cnt_ref, xp_ref, carry):
    i = pl.program_id(0)

    @pl.when(i == 0)
    def _init():
        carry[...] = jnp.zeros_like(carry)

    xb = x_ref[...]
    xp_ref[...] = _pack_bf16(xb.astype(BF16))
    logits = jnp.dot(xb, wg_ref[...], preferred_element_type=F32)
    colf = lax.broadcasted_iota(jnp.int32, (RB, EP), 1).astype(F32)
    valid = colf < float(N_EXP)
    logits = jnp.where(valid, logits, -1e30)
    mx = jnp.max(logits, axis=1, keepdims=True)
    ex = jnp.exp(logits - mx)
    probs = ex / jnp.sum(ex, axis=1, keepdims=True)
    v1 = jnp.max(jnp.where(valid, probs, -1.0), axis=1, keepdims=True)
    i1 = jnp.min(jnp.where(valid & (probs == v1), colf, 127.0),
                 axis=1, keepdims=True)
    oh1 = colf == i1
    probs2 = jnp.where(oh1 | ~valid, -1.0, probs)
    v2 = jnp.max(probs2, axis=1, keepdims=True)
    i2 = jnp.min(jnp.where(probs2 == v2, colf, 127.0), axis=1, keepdims=True)
    oh2 = colf == i2

    m = (oh1 | oh2).astype(F32)
    rowi = lax.broadcasted_iota(jnp.int32, (RB, RB), 0)
    coli = lax.broadcasted_iota(jnp.int32, (RB, RB), 1)
    tril = (coli < rowi).astype(F32)
    within = jnp.dot(tril, m, preferred_element_type=F32)
    rank = within + carry[...]
    r1 = jnp.sum(jnp.where(oh1, rank, 0.0), axis=1, keepdims=True)
    r2 = jnp.sum(jnp.where(oh2, rank, 0.0), axis=1, keepdims=True)
    carry[...] = carry[...] + jnp.sum(m, axis=0, keepdims=True)

    coli2 = lax.broadcasted_iota(jnp.int32, (RB, EP), 1)
    meta = jnp.where(coli2 == 0, i1, 0.0)
    meta = jnp.where(coli2 == 1, i2, meta)
    meta = jnp.where(coli2 == 2, v1, meta)
    meta = jnp.where(coli2 == 3, v2, meta)
    meta = jnp.where(coli2 == 4, r1, meta)
    meta = jnp.where(coli2 == 5, r2, meta)
    meta_ref[...] = meta
    cnt_ref[...] = jnp.broadcast_to(carry[...], (8, EP))


def _router_call(x2, wg_pad):
    return pl.pallas_call(
        _router_body,
        grid=(N_RB,),
        in_specs=[
            pl.BlockSpec((RB, D_MODEL), lambda i: (i, 0)),
            pl.BlockSpec((D_MODEL, EP), lambda i: (0, 0)),
        ],
        out_specs=[pl.BlockSpec((RB, EP), lambda i: (i, 0)),
                   pl.BlockSpec((8, EP), lambda i: (0, 0)),
                   pl.BlockSpec((RB, DH), lambda i: (i, 0))],
        out_shape=[jax.ShapeDtypeStruct((N_TOK, EP), F32),
                   jax.ShapeDtypeStruct((8, EP), F32),
                   jax.ShapeDtypeStruct((N_TOK, DH), I32)],
        scratch_shapes=[pltpu.VMEM((1, EP), F32)],
    )(x2, wg_pad)


def _finalize_body(cnt_ref, meta_ref, pos1_ref, pos2_ref, te_ref):
    cnt = cnt_ref[0:1, :]
    padded = jnp.ceil(cnt / float(M_TILE)) * float(M_TILE)
    rowi = lax.broadcasted_iota(jnp.int32, (EP, EP), 0)
    coli = lax.broadcasted_iota(jnp.int32, (EP, EP), 1)
    upper = (rowi < coli).astype(F32)
    starts = jnp.dot(padded, upper, preferred_element_type=F32)
    ends_t = (starts + padded) / float(M_TILE)
    validE = coli < N_EXP
    en_b = jnp.broadcast_to(ends_t, (EP, EP))
    cnt_le = jnp.sum(jnp.where(validE & (en_b <= rowi.astype(F32)), 1.0, 0.0),
                     axis=1, keepdims=True)
    te = jnp.minimum(cnt_le, float(N_EXP - 1))
    te_ref[...] = jnp.broadcast_to(te, (EP, EP))
    colt = lax.broadcasted_iota(jnp.int32, (N_TOK, EP), 1).astype(F32)
    starts_b = jnp.broadcast_to(starts, (N_TOK, EP))
    e1 = meta_ref[:, 0:1]
    e2 = meta_ref[:, 1:2]
    r1 = meta_ref[:, 4:5]
    r2 = meta_ref[:, 5:6]
    s1 = jnp.sum(jnp.where(colt == e1, starts_b, 0.0), axis=1, keepdims=True)
    s2 = jnp.sum(jnp.where(colt == e2, starts_b, 0.0), axis=1, keepdims=True)
    pos1_ref[...] = jnp.broadcast_to(s1 + r1, (N_TOK, EP))
    pos2_ref[...] = jnp.broadcast_to(s2 + r2, (N_TOK, EP))


def _finalize_call(cnt, meta):
    full = lambda shp: pl.BlockSpec(shp, lambda: (0, 0))
    return pl.pallas_call(
        _finalize_body,
        in_specs=[full((8, EP)), full((N_TOK, EP))],
        out_specs=[full((N_TOK, EP)), full((N_TOK, EP)), full((EP, EP))],
        out_shape=[jax.ShapeDtypeStruct((N_TOK, EP), F32),
                   jax.ShapeDtypeStruct((N_TOK, EP), F32),
                   jax.ShapeDtypeStruct((EP, EP), F32)],
    )(cnt, meta)


def _grouped_ffn_body(te_ref, xs_ref, wg_ref, wu_ref, wd_ref, ys_ref):
    e = te_ref[pl.program_id(0)]
    xb = _unpack_bf16(xs_ref[...])
    g = jnp.dot(xb, wg_ref[e], preferred_element_type=F32)
    u = jnp.dot(xb, wu_ref[e], preferred_element_type=F32)
    h = (g * jax.nn.sigmoid(g) * u).astype(BF16)
    y = jnp.dot(h, wd_ref[e], preferred_element_type=F32)
    ys_ref[...] = _pack_bf16(y.astype(BF16))


def _grouped_ffn_call(te, xs, wg_e, wu_e, wd_e):
    grid_spec = pltpu.PrefetchScalarGridSpec(
        num_scalar_prefetch=1,
        grid=(N_TILES,),
        in_specs=[
            pl.BlockSpec((M_TILE, DH), lambda i, te: (i, 0)),
            pl.BlockSpec((N_EXP, D_MODEL, D_EXP), lambda i, te: (0, 0, 0)),
            pl.BlockSpec((N_EXP, D_MODEL, D_EXP), lambda i, te: (0, 0, 0)),
            pl.BlockSpec((N_EXP, D_EXP, D_MODEL), lambda i, te: (0, 0, 0)),
        ],
        out_specs=pl.BlockSpec((M_TILE, DH), lambda i, te: (i, 0)),
    )
    return pl.pallas_call(
        _grouped_ffn_body,
        grid_spec=grid_spec,
        out_shape=jax.ShapeDtypeStruct((N_SLOT, DH), I32),
    )(te, xs, wg_e, wu_e, wd_e)


def _shared_combine_body(x_ref, wgs_ref, wus_ref, wds_ref,
                         y1_ref, y2_ref, meta_ref, out_ref):
    xb = _unpack_bf16(x_ref[...])
    g = jnp.dot(xb, wgs_ref[...], preferred_element_type=F32)
    u = jnp.dot(xb, wus_ref[...], preferred_element_type=F32)
    h = (g * jax.nn.sigmoid(g) * u).astype(BF16)
    sh = jnp.dot(h, wds_ref[...], preferred_element_type=F32)
    w1 = meta_ref[:, 2:3]
    w2 = meta_ref[:, 3:4]
    y1 = _unpack_bf16(y1_ref[...]).astype(F32)
    y2 = _unpack_bf16(y2_ref[...]).astype(F32)
    out_ref[...] = sh + w1 * y1 + w2 * y2


def _shared_combine_call(xp, wg_s, wu_s, wd_s, y1, y2, meta):
    return pl.pallas_call(
        _shared_combine_body,
        grid=(N_CB,),
        in_specs=[
            pl.BlockSpec((CB, DH), lambda i: (i, 0)),
            pl.BlockSpec((D_MODEL, D_SHARED), lambda i: (0, 0)),
            pl.BlockSpec((D_MODEL, D_SHARED), lambda i: (0, 0)),
            pl.BlockSpec((D_SHARED, D_MODEL), lambda i: (0, 0)),
            pl.BlockSpec((CB, DH), lambda i: (i, 0)),
            pl.BlockSpec((CB, DH), lambda i: (i, 0)),
            pl.BlockSpec((CB, EP), lambda i: (i, 0)),
        ],
        out_specs=pl.BlockSpec((CB, D_MODEL), lambda i: (i, 0)),
        out_shape=jax.ShapeDtypeStruct((N_TOK, D_MODEL), F32),
    )(xp, wg_s, wu_s, wd_s, y1, y2, meta)


def _scatter_x_call(xp, pos1, pos2):
    mesh = plsc.VectorSubcoreMesh(core_axis_name="c", subcore_axis_name="s")

    @functools.partial(
        pl.kernel,
        out_type=jax.ShapeDtypeStruct((N_SLOT, DH), I32),
        mesh=mesh,
        scratch_types=[
            pltpu.VMEM((TPW, DH), I32),
            pltpu.VMEM((TPW,), jnp.int32),
            pltpu.VMEM((TPW,), jnp.int32),
            pltpu.SemaphoreType.DMA,
        ],
    )
    def _scatter(x_hbm, p1_hbm, p2_hbm, xs_hbm, xbuf, i1v, i2v, sem):
        wid = lax.axis_index("s") * NUM_SC_CORES + lax.axis_index("c")
        base = wid * TPW
        pltpu.sync_copy(x_hbm.at[pl.ds(base, TPW)], xbuf)
        pltpu.sync_copy(p1_hbm.at[pl.ds(base, TPW)], i1v)
        pltpu.sync_copy(p2_hbm.at[pl.ds(base, TPW)], i2v)
        pltpu.async_copy(xbuf, xs_hbm.at[i1v], sem).wait()
        pltpu.async_copy(xbuf, xs_hbm.at[i2v], sem).wait()

    return _scatter(xp, pos1, pos2)


def _gather_ys_call(ys, pos1, pos2):
    mesh = plsc.VectorSubcoreMesh(core_axis_name="c", subcore_axis_name="s")

    @functools.partial(
        pl.kernel,
        out_type=(jax.ShapeDtypeStruct((N_TOK, DH), I32),
                  jax.ShapeDtypeStruct((N_TOK, DH), I32)),
        mesh=mesh,
        scratch_types=[
            pltpu.VMEM((TPW, DH), I32),
            pltpu.VMEM((TPW, DH), I32),
            pltpu.VMEM((TPW,), jnp.int32),
            pltpu.VMEM((TPW,), jnp.int32),
            pltpu.SemaphoreType.DMA,
        ],
    )
    def _gather(ys_hbm, p1_hbm, p2_hbm, y1_hbm, y2_hbm, buf1, buf2,
                i1v, i2v, sem):
        wid = lax.axis_index("s") * NUM_SC_CORES + lax.axis_index("c")
        base = wid * TPW
        pltpu.sync_copy(p1_hbm.at[pl.ds(base, TPW)], i1v)
        pltpu.sync_copy(p2_hbm.at[pl.ds(base, TPW)], i2v)
        c1 = pltpu.async_copy(ys_hbm.at[i1v], buf1, sem)
        c2 = pltpu.async_copy(ys_hbm.at[i2v], buf2, sem)
        c1.wait()
        pltpu.sync_copy(buf1, y1_hbm.at[pl.ds(base, TPW)])
        c2.wait()
        pltpu.sync_copy(buf2, y2_hbm.at[pl.ds(base, TPW)])

    return _gather(ys, pos1, pos2)


def kernel(x, W_g, Wg_e, Wu_e, Wd_e, Wg_s, Wu_s, Wd_s):
    b, t, d = x.shape
    x2 = x.reshape(b * t, d)
    wg_pad = jnp.zeros((D_MODEL, EP), F32).at[:, :N_EXP].set(W_g)

    meta, cnt, xp = _router_call(x2, wg_pad)
    pos1b, pos2b, teb = _finalize_call(cnt, meta)

    pos1 = pos1b[:, 0].astype(jnp.int32)
    pos2 = pos2b[:, 0].astype(jnp.int32)
    te = teb[:N_TILES, 0].astype(jnp.int32)

    xs = _scatter_x_call(xp, pos1, pos2)
    ys = _grouped_ffn_call(te, xs, Wg_e.astype(BF16), Wu_e.astype(BF16),
                           Wd_e.astype(BF16))
    y1, y2 = _gather_ys_call(ys, pos1, pos2)
    out = _shared_combine_call(xp, Wg_s.astype(BF16), Wu_s.astype(BF16),
                               Wd_s.astype(BF16), y1, y2, meta)
    return out.reshape(b, t, d)


# speedup vs baseline: 1.0375x; 1.0375x over previous
import functools

import jax
import jax.numpy as jnp
from jax import lax
from jax.experimental import pallas as pl
from jax.experimental.pallas import tpu as pltpu
from jax.experimental.pallas import tpu_sc as plsc

F32 = jnp.float32
BF16 = jnp.bfloat16
I32 = jnp.int32

N_TOK = 2048
D_MODEL = 1024
DH = D_MODEL // 2
N_EXP = 8
D_EXP = 256
D_SHARED = 512
EP = 128
M_TILE = 512
N_TILES = 16
N_SLOT = N_TILES * M_TILE
RB = 512
N_RB = N_TOK // RB
CB = 512
N_CB = N_TOK // CB
NUM_SC_CORES = 2
NUM_SC_SUBCORES = 16
N_WORKERS = NUM_SC_CORES * NUM_SC_SUBCORES
TPW = N_TOK // N_WORKERS


def _pack_bf16(v):
    lo = lax.bitcast_convert_type(v[:, :DH], jnp.int16).astype(I32) & 0xFFFF
    hi = lax.bitcast_convert_type(v[:, DH:], jnp.int16).astype(I32)
    return lax.shift_left(hi, 16) | lo


def _unpack_bf16(p):
    lo = lax.bitcast_convert_type(p.astype(jnp.int16), BF16)
    hi = lax.bitcast_convert_type(
        lax.shift_right_logical(p, 16).astype(jnp.int16), BF16)
    return jnp.concatenate([lo, hi], axis=1)


def _router_body(x_ref, wg_ref, meta_ref, cnt_ref, xp_ref, carry):
    i = pl.program_id(0)

    @pl.when(i == 0)
    def _init():
        carry[...] = jnp.zeros_like(carry)

    xb = x_ref[...]
    xp_ref[...] = _pack_bf16(xb.astype(BF16))
    logits = jnp.dot(xb, wg_ref[...], preferred_element_type=F32)
    colf = lax.broadcasted_iota(jnp.int32, (RB, EP), 1).astype(F32)
    valid = colf < float(N_EXP)
    logits = jnp.where(valid, logits, -1e30)
    mx = jnp.max(logits, axis=1, keepdims=True)
    ex = jnp.exp(logits - mx)
    probs = ex / jnp.sum(ex, axis=1, keepdims=True)
    v1 = jnp.max(jnp.where(valid, probs, -1.0), axis=1, keepdims=True)
    i1 = jnp.min(jnp.where(valid & (probs == v1), colf, 127.0),
                 axis=1, keepdims=True)
    oh1 = colf == i1
    probs2 = jnp.where(oh1 | ~valid, -1.0, probs)
    v2 = jnp.max(probs2, axis=1, keepdims=True)
    i2 = jnp.min(jnp.where(probs2 == v2, colf, 127.0), axis=1, keepdims=True)
    oh2 = colf == i2

    m = (oh1 | oh2).astype(F32)
    rowi = lax.broadcasted_iota(jnp.int32, (RB, RB), 0)
    coli = lax.broadcasted_iota(jnp.int32, (RB, RB), 1)
    tril = (coli < rowi).astype(F32)
    within = jnp.dot(tril, m, preferred_element_type=F32)
    rank = within + carry[...]
    r1 = jnp.sum(jnp.where(oh1, rank, 0.0), axis=1, keepdims=True)
    r2 = jnp.sum(jnp.where(oh2, rank, 0.0), axis=1, keepdims=True)
    carry[...] = carry[...] + jnp.sum(m, axis=0, keepdims=True)

    coli2 = lax.broadcasted_iota(jnp.int32, (RB, EP), 1)
    meta = jnp.where(coli2 == 0, i1, 0.0)
    meta = jnp.where(coli2 == 1, i2, meta)
    meta = jnp.where(coli2 == 2, v1, meta)
    meta = jnp.where(coli2 == 3, v2, meta)
    meta = jnp.where(coli2 == 4, r1, meta)
    meta = jnp.where(coli2 == 5, r2, meta)
    meta_ref[...] = meta
    cnt_ref[...] = jnp.broadcast_to(carry[...], (8, EP))


def _router_call(x2, wg_pad):
    return pl.pallas_call(
        _router_body,
        grid=(N_RB,),
        in_specs=[
            pl.BlockSpec((RB, D_MODEL), lambda i: (i, 0)),
            pl.BlockSpec((D_MODEL, EP), lambda i: (0, 0)),
        ],
        out_specs=[pl.BlockSpec((RB, EP), lambda i: (i, 0)),
                   pl.BlockSpec((8, EP), lambda i: (0, 0)),
                   pl.BlockSpec((RB, DH), lambda i: (i, 0))],
        out_shape=[jax.ShapeDtypeStruct((N_TOK, EP), F32),
                   jax.ShapeDtypeStruct((8, EP), F32),
                   jax.ShapeDtypeStruct((N_TOK, DH), I32)],
        scratch_shapes=[pltpu.VMEM((1, EP), F32)],
    )(x2, wg_pad)


def _finalize_body(cnt_ref, meta_ref, pos1_ref, pos2_ref, te_ref):
    cnt = cnt_ref[0:1, :]
    padded = jnp.ceil(cnt / float(M_TILE)) * float(M_TILE)
    rowi = lax.broadcasted_iota(jnp.int32, (EP, EP), 0)
    coli = lax.broadcasted_iota(jnp.int32, (EP, EP), 1)
    upper = (rowi < coli).astype(F32)
    starts = jnp.dot(padded, upper, preferred_element_type=F32)
    ends_t = (starts + padded) / float(M_TILE)
    validE = coli < N_EXP
    en_b = jnp.broadcast_to(ends_t, (EP, EP))
    cnt_le = jnp.sum(jnp.where(validE & (en_b <= rowi.astype(F32)), 1.0, 0.0),
                     axis=1, keepdims=True)
    te = jnp.minimum(cnt_le, float(N_EXP - 1))
    te_ref[...] = jnp.broadcast_to(te, (EP, EP))
    colt = lax.broadcasted_iota(jnp.int32, (N_TOK, EP), 1).astype(F32)
    starts_b = jnp.broadcast_to(starts, (N_TOK, EP))
    e1 = meta_ref[:, 0:1]
    e2 = meta_ref[:, 1:2]
    r1 = meta_ref[:, 4:5]
    r2 = meta_ref[:, 5:6]
    s1 = jnp.sum(jnp.where(colt == e1, starts_b, 0.0), axis=1, keepdims=True)
    s2 = jnp.sum(jnp.where(colt == e2, starts_b, 0.0), axis=1, keepdims=True)
    pos1_ref[...] = jnp.broadcast_to(s1 + r1, (N_TOK, EP))
    pos2_ref[...] = jnp.broadcast_to(s2 + r2, (N_TOK, EP))


def _finalize_call(cnt, meta):
    full = lambda shp: pl.BlockSpec(shp, lambda: (0, 0))
    return pl.pallas_call(
        _finalize_body,
        in_specs=[full((8, EP)), full((N_TOK, EP))],
        out_specs=[full((N_TOK, EP)), full((N_TOK, EP)), full((EP, EP))],
        out_shape=[jax.ShapeDtypeStruct((N_TOK, EP), F32),
                   jax.ShapeDtypeStruct((N_TOK, EP), F32),
                   jax.ShapeDtypeStruct((EP, EP), F32)],
    )(cnt, meta)


def _grouped_ffn_body(te_ref, xs_ref, wg_ref, wu_ref, wd_ref, ys_ref):
    e = te_ref[pl.program_id(0)]
    xb = _unpack_bf16(xs_ref[...])
    g = jnp.dot(xb, wg_ref[e], preferred_element_type=F32)
    u = jnp.dot(xb, wu_ref[e], preferred_element_type=F32)
    h = (g * jax.nn.sigmoid(g) * u).astype(BF16)
    y = jnp.dot(h, wd_ref[e], preferred_element_type=F32)
    ys_ref[...] = _pack_bf16(y.astype(BF16))


def _grouped_ffn_call(te, xs, wg_e, wu_e, wd_e):
    grid_spec = pltpu.PrefetchScalarGridSpec(
        num_scalar_prefetch=1,
        grid=(N_TILES,),
        in_specs=[
            pl.BlockSpec((M_TILE, DH), lambda i, te: (i, 0)),
            pl.BlockSpec((N_EXP, D_MODEL, D_EXP), lambda i, te: (0, 0, 0)),
            pl.BlockSpec((N_EXP, D_MODEL, D_EXP), lambda i, te: (0, 0, 0)),
            pl.BlockSpec((N_EXP, D_EXP, D_MODEL), lambda i, te: (0, 0, 0)),
        ],
        out_specs=pl.BlockSpec((M_TILE, DH), lambda i, te: (i, 0)),
    )
    return pl.pallas_call(
        _grouped_ffn_body,
        grid_spec=grid_spec,
        out_shape=jax.ShapeDtypeStruct((N_SLOT, DH), I32),
    )(te, xs, wg_e, wu_e, wd_e)


def _shared_combine_body(x_ref, wgs_ref, wus_ref, wds_ref,
                         y1_ref, y2_ref, meta_ref, out_ref):
    xb = _unpack_bf16(x_ref[...])
    g = jnp.dot(xb, wgs_ref[...], preferred_element_type=F32)
    u = jnp.dot(xb, wus_ref[...], preferred_element_type=F32)
    h = (g * jax.nn.sigmoid(g) * u).astype(BF16)
    sh = jnp.dot(h, wds_ref[...], preferred_element_type=F32)
    w1 = meta_ref[:, 2:3]
    w2 = meta_ref[:, 3:4]
    y1 = _unpack_bf16(y1_ref[...]).astype(F32)
    y2 = _unpack_bf16(y2_ref[...]).astype(F32)
    out_ref[...] = sh + w1 * y1 + w2 * y2


def _shared_combine_call(xp, wg_s, wu_s, wd_s, y1, y2, meta):
    return pl.pallas_call(
        _shared_combine_body,
        grid=(N_CB,),
        in_specs=[
            pl.BlockSpec((CB, DH), lambda i: (i, 0)),
            pl.BlockSpec((D_MODEL, D_SHARED), lambda i: (0, 0)),
            pl.BlockSpec((D_MODEL, D_SHARED), lambda i: (0, 0)),
            pl.BlockSpec((D_SHARED, D_MODEL), lambda i: (0, 0)),
            pl.BlockSpec((CB, DH), lambda i: (i, 0)),
            pl.BlockSpec((CB, DH), lambda i: (i, 0)),
            pl.BlockSpec((CB, EP), lambda i: (i, 0)),
        ],
        out_specs=pl.BlockSpec((CB, D_MODEL), lambda i: (i, 0)),
        out_shape=jax.ShapeDtypeStruct((N_TOK, D_MODEL), F32),
    )(xp, wg_s, wu_s, wd_s, y1, y2, meta)


def _scatter_x_call(xp, pos1, pos2):
    mesh = plsc.VectorSubcoreMesh(core_axis_name="c", subcore_axis_name="s")

    @functools.partial(
        pl.kernel,
        out_type=jax.ShapeDtypeStruct((N_SLOT, DH), I32),
        mesh=mesh,
        scratch_types=[
            pltpu.VMEM((TPW, DH), I32),
            pltpu.VMEM((TPW,), jnp.int32),
            pltpu.VMEM((TPW,), jnp.int32),
            pltpu.SemaphoreType.DMA,
        ],
    )
    def _scatter(x_hbm, p1_hbm, p2_hbm, xs_hbm, xbuf, i1v, i2v, sem):
        wid = lax.axis_index("s") * NUM_SC_CORES + lax.axis_index("c")
        base = wid * TPW
        pltpu.sync_copy(x_hbm.at[pl.ds(base, TPW)], xbuf)
        pltpu.sync_copy(p1_hbm.at[pl.ds(base, TPW)], i1v)
        pltpu.sync_copy(p2_hbm.at[pl.ds(base, TPW)], i2v)
        pltpu.async_copy(xbuf, xs_hbm.at[i1v], sem).wait()
        pltpu.async_copy(xbuf, xs_hbm.at[i2v], sem).wait()

    return _scatter(xp, pos1, pos2)


def _gather_ys_call(ys, pos1, pos2):
    mesh = plsc.VectorSubcoreMesh(core_axis_name="c", subcore_axis_name="s")

    @functools.partial(
        pl.kernel,
        out_type=(jax.ShapeDtypeStruct((N_TOK, DH), I32),
                  jax.ShapeDtypeStruct((N_TOK, DH), I32)),
        mesh=mesh,
        scratch_types=[
            pltpu.VMEM((TPW, DH), I32),
            pltpu.VMEM((TPW, DH), I32),
            pltpu.VMEM((TPW,), jnp.int32),
            pltpu.VMEM((TPW,), jnp.int32),
            pltpu.SemaphoreType.DMA,
        ],
    )
    def _gather(ys_hbm, p1_hbm, p2_hbm, y1_hbm, y2_hbm, buf1, buf2,
                i1v, i2v, sem):
        wid = lax.axis_index("s") * NUM_SC_CORES + lax.axis_index("c")
        base = wid * TPW
        pltpu.sync_copy(p1_hbm.at[pl.ds(base, TPW)], i1v)
        pltpu.sync_copy(p2_hbm.at[pl.ds(base, TPW)], i2v)
        c1 = pltpu.async_copy(ys_hbm.at[i1v], buf1, sem)
        c2 = pltpu.async_copy(ys_hbm.at[i2v], buf2, sem)
        c1.wait()
        pltpu.sync_copy(buf1, y1_hbm.at[pl.ds(base, TPW)])
        c2.wait()
        pltpu.sync_copy(buf2, y2_hbm.at[pl.ds(base, TPW)])

    return _gather(ys, pos1, pos2)


def kernel(x, W_g, Wg_e, Wu_e, Wd_e, Wg_s, Wu_s, Wd_s):
    b, t, d = x.shape
    x2 = x.reshape(b * t, d)
    wg_pad = jnp.zeros((D_MODEL, EP), F32).at[:, :N_EXP].set(W_g)

    meta, cnt, xp = _router_call(x2, wg_pad)
    pos1b, pos2b, teb = _finalize_call(cnt, meta)

    pos1 = pos1b[:, 0].astype(jnp.int32)
    pos2 = pos2b[:, 0].astype(jnp.int32)
    te = teb[:N_TILES, 0].astype(jnp.int32)

    xs = _scatter_x_call(xp, pos1, pos2)
    ys = _grouped_ffn_call(te, xs, Wg_e.astype(BF16), Wu_e.astype(BF16),
                           Wd_e.astype(BF16))
    y1, y2 = _gather_ys_call(ys, pos1, pos2)
    out = _shared_combine_call(xp, Wg_s.astype(BF16), Wu_s.astype(BF16),
                               Wd_s.astype(BF16), y1, y2, meta)
    return out.reshape(b, t, d)
